# baseline (speedup 1.0000x reference)
import functools

import jax
import jax.numpy as jnp
import numpy as np
from jax import lax
from jax.experimental import pallas as pl
from jax.experimental.pallas import tpu as pltpu
from jax.experimental.pallas import tpu_sc as plsc

N, D = 16384, 100
NC, NS = 2, 16
NW = NC * NS
NG = N // 16
UNROLL = 16
NCOL1 = 96
TAIL = N // 8
BL = 2048


def _threefry2x32(k1, k2, x1, x2):
    rotations = [(13, 15, 26, 6), (17, 29, 16, 24)]
    ks = [np.uint32(k1), np.uint32(k2),
          np.uint32(np.uint32(k1) ^ np.uint32(k2) ^ np.uint32(0x1BD11BDA))]

    def rotl(v, d):
        return (v << np.uint32(d)) | (v >> np.uint32(32 - d))

    x1 = (x1 + ks[0]).astype(np.uint32)
    x2 = (x2 + ks[1]).astype(np.uint32)
    for r in range(5):
        for d in rotations[r % 2]:
            x1 = (x1 + x2).astype(np.uint32)
            x2 = rotl(x2, d)
            x2 = x1 ^ x2
        x1 = (x1 + ks[(r + 1) % 3]).astype(np.uint32)
        x2 = (x2 + ks[(r + 2) % 3] + np.uint32(r + 1)).astype(np.uint32)
    return x1, x2


def _random_bits(keypair, n):
    k1, k2 = keypair
    b1, b2 = _threefry2x32(k1, k2, np.zeros(n, dtype=np.uint32),
                           np.arange(n, dtype=np.uint32))
    return b1 ^ b2


def _split(keypair):
    b1, b2 = _threefry2x32(*keypair, np.zeros(2, dtype=np.uint32),
                           np.arange(2, dtype=np.uint32))
    return (b1[0], b2[0]), (b1[1], b2[1])


@functools.lru_cache(maxsize=1)
def _rng_consts():
    kb, kp = _split((np.uint32(0), np.uint32(42)))
    bits = _random_bits(kb, N * D)
    u = ((bits >> np.uint32(9)) | np.uint32(0x3F800000)).view(np.float32)
    u = (u - np.float32(1.0)).reshape(N, D)
    perm = np.arange(N, dtype=np.int32)
    num_rounds = int(np.ceil(3 * np.log(N) / np.log(2.0**32 - 1)))
    key = kp
    for _ in range(num_rounds):
        key, subkey = _split(key)
        sort_keys = _random_bits(subkey, N)
        perm = perm[np.argsort(sort_keys, kind="stable")]
    return np.ascontiguousarray(u.T), perm


def _sc_gather_body(xt_hbm, perm_hbm, out_hbm, perm_v,
                    xc0_v, xc1_v, og0_v, og1_v,
                    si0, si1, so0, so1):
    wid = lax.axis_index("s") * NC + lax.axis_index("c")
    j2 = NCOL1 + (wid >> 3)
    rlo = (wid & 7) * TAIL
    cols = [wid * (NCOL1 // NW) + t for t in range(NCOL1 // NW)] + [j2]
    xc = [xc0_v, xc1_v]
    og = [og0_v, og1_v]
    sin = [si0, si1]
    sout = [so0, so1]

    in_h = {0: pltpu.async_copy(xt_hbm.at[cols[0]], xc[0], sin[0])}
    pltpu.sync_copy(perm_hbm, perm_v)
    out_h = {}
    ntask = len(cols)
    for t in range(ntask):
        b = t % 2
        in_h.pop(t).wait()
        if t + 1 < ntask:
            in_h[t + 1] = pltpu.async_copy(xt_hbm.at[cols[t + 1]],
                                           xc[(t + 1) % 2], sin[(t + 1) % 2])
        if t >= 2:
            out_h.pop(t - 2).wait()
        xc_v = xc[b]
        og_v = og[b]
        if t < ntask - 1:
            @plsc.parallel_loop(0, NG, unroll=UNROLL)
            def _(g):
                f0 = g * 16
                idx = perm_v[pl.ds(f0, 16)]
                og_v[pl.ds(f0, 16)] = plsc.load_gather(xc_v, [idx])

            out_h[t] = pltpu.async_copy(og_v, out_hbm.at[cols[t]], sout[b])
        else:
            @plsc.parallel_loop(0, TAIL // 16, unroll=UNROLL)
            def _(g):
                f0 = g * 16
                idx = perm_v[pl.ds(rlo + f0, 16)]
                og_v[pl.ds(f0, 16)] = plsc.load_gather(xc_v, [idx])

            out_h[t] = pltpu.async_copy(og_v.at[pl.ds(0, TAIL)],
                                        out_hbm.at[j2, pl.ds(rlo, TAIL)],
                                        sout[b])
    for h in out_h.values():
        h.wait()


def _tc_select_body(u_ref, p_ref, xg_ref, xo_ref, o_ref):
    mask = u_ref[...] < p_ref[...][None, :]
    o_ref[...] = jnp.where(mask, xg_ref[...], xo_ref[...])


def kernel(x, p):
    ut, perm = _rng_consts()
    xt = x.T
    mesh = plsc.VectorSubcoreMesh(core_axis_name="c", subcore_axis_name="s",
                                  num_cores=NC, num_subcores=NS)
    gather_run = pl.kernel(
        _sc_gather_body,
        out_type=jax.ShapeDtypeStruct((D, N), jnp.float32),
        mesh=mesh,
        scratch_types=[
            pltpu.VMEM((N,), jnp.int32),
            pltpu.VMEM((N,), jnp.float32),
            pltpu.VMEM((N,), jnp.float32),
            pltpu.VMEM((N,), jnp.float32),
            pltpu.VMEM((N,), jnp.float32),
            pltpu.SemaphoreType.DMA,
            pltpu.SemaphoreType.DMA,
            pltpu.SemaphoreType.DMA,
            pltpu.SemaphoreType.DMA,
        ],
        compiler_params=pltpu.CompilerParams(
            use_tc_tiling_on_sc=True, needs_layout_passes=False),
    )
    xg_t = gather_run(xt, jnp.asarray(perm))

    out_t = pl.pallas_call(
        _tc_select_body,
        out_shape=jax.ShapeDtypeStruct((D, N), jnp.float32),
        grid=(N // BL,),
        in_specs=[
            pl.BlockSpec((D, BL), lambda i: (0, i)),
            pl.BlockSpec((BL,), lambda i: (i,)),
            pl.BlockSpec((D, BL), lambda i: (0, i)),
            pl.BlockSpec((D, BL), lambda i: (0, i)),
        ],
        out_specs=pl.BlockSpec((D, BL), lambda i: (0, i)),
        input_output_aliases={2: 0},
    )(jnp.asarray(ut), p, xg_t, xt)
    return out_t.T


# speedup vs baseline: 1.0078x; 1.0078x over previous
import functools

import jax
import jax.numpy as jnp
import numpy as np
from jax import lax
from jax.experimental import pallas as pl
from jax.experimental.pallas import tpu as pltpu
from jax.experimental.pallas import tpu_sc as plsc

N, D = 16384, 100
NC, NS = 2, 16
NW = NC * NS
NG = N // 16
UNROLL = 8
NCOL1 = 96
TAIL = N // 8
BL = 2048


def _threefry2x32(k1, k2, x1, x2):
    rotations = [(13, 15, 26, 6), (17, 29, 16, 24)]
    ks = [np.uint32(k1), np.uint32(k2),
          np.uint32(np.uint32(k1) ^ np.uint32(k2) ^ np.uint32(0x1BD11BDA))]

    def rotl(v, d):
        return (v << np.uint32(d)) | (v >> np.uint32(32 - d))

    x1 = (x1 + ks[0]).astype(np.uint32)
    x2 = (x2 + ks[1]).astype(np.uint32)
    for r in range(5):
        for d in rotations[r % 2]:
            x1 = (x1 + x2).astype(np.uint32)
            x2 = rotl(x2, d)
            x2 = x1 ^ x2
        x1 = (x1 + ks[(r + 1) % 3]).astype(np.uint32)
        x2 = (x2 + ks[(r + 2) % 3] + np.uint32(r + 1)).astype(np.uint32)
    return x1, x2


def _random_bits(keypair, n):
    k1, k2 = keypair
    b1, b2 = _threefry2x32(k1, k2, np.zeros(n, dtype=np.uint32),
                           np.arange(n, dtype=np.uint32))
    return b1 ^ b2


def _split(keypair):
    b1, b2 = _threefry2x32(*keypair, np.zeros(2, dtype=np.uint32),
                           np.arange(2, dtype=np.uint32))
    return (b1[0], b2[0]), (b1[1], b2[1])


@functools.lru_cache(maxsize=1)
def _rng_consts():
    kb, kp = _split((np.uint32(0), np.uint32(42)))
    bits = _random_bits(kb, N * D)
    u = ((bits >> np.uint32(9)) | np.uint32(0x3F800000)).view(np.float32)
    u = (u - np.float32(1.0)).reshape(N, D)
    perm = np.arange(N, dtype=np.int32)
    num_rounds = int(np.ceil(3 * np.log(N) / np.log(2.0**32 - 1)))
    key = kp
    for _ in range(num_rounds):
        key, subkey = _split(key)
        sort_keys = _random_bits(subkey, N)
        perm = perm[np.argsort(sort_keys, kind="stable")]
    return np.ascontiguousarray(u.T), perm


def _sc_gather_body(xt_hbm, perm_hbm, out_hbm, perm_v,
                    xc0_v, xc1_v, og0_v, og1_v,
                    si0, si1, so0, so1):
    wid = lax.axis_index("s") * NC + lax.axis_index("c")
    j2 = NCOL1 + (wid >> 3)
    rlo = (wid & 7) * TAIL
    cols = [wid * (NCOL1 // NW) + t for t in range(NCOL1 // NW)] + [j2]
    xc = [xc0_v, xc1_v]
    og = [og0_v, og1_v]
    sin = [si0, si1]
    sout = [so0, so1]

    in_h = {0: pltpu.async_copy(xt_hbm.at[cols[0]], xc[0], sin[0])}
    pltpu.sync_copy(perm_hbm, perm_v)
    out_h = {}
    ntask = len(cols)
    for t in range(ntask):
        b = t % 2
        in_h.pop(t).wait()
        if t + 1 < ntask:
            in_h[t + 1] = pltpu.async_copy(xt_hbm.at[cols[t + 1]],
                                           xc[(t + 1) % 2], sin[(t + 1) % 2])
        if t >= 2:
            out_h.pop(t - 2).wait()
        xc_v = xc[b]
        og_v = og[b]
        if t < ntask - 1:
            @plsc.parallel_loop(0, NG, unroll=UNROLL)
            def _(g):
                f0 = g * 16
                idx = perm_v[pl.ds(f0, 16)]
                og_v[pl.ds(f0, 16)] = plsc.load_gather(xc_v, [idx])

            out_h[t] = pltpu.async_copy(og_v, out_hbm.at[cols[t]], sout[b])
        else:
            @plsc.parallel_loop(0, TAIL // 16, unroll=UNROLL)
            def _(g):
                f0 = g * 16
                idx = perm_v[pl.ds(rlo + f0, 16)]
                og_v[pl.ds(f0, 16)] = plsc.load_gather(xc_v, [idx])

            out_h[t] = pltpu.async_copy(og_v.at[pl.ds(0, TAIL)],
                                        out_hbm.at[j2, pl.ds(rlo, TAIL)],
                                        sout[b])
    for h in out_h.values():
        h.wait()


def _tc_select_body(u_ref, p_ref, xg_ref, xo_ref, o_ref):
    mask = u_ref[...] < p_ref[...][None, :]
    o_ref[...] = jnp.where(mask, xg_ref[...], xo_ref[...])


def kernel(x, p):
    ut, perm = _rng_consts()
    xt = x.T
    mesh = plsc.VectorSubcoreMesh(core_axis_name="c", subcore_axis_name="s",
                                  num_cores=NC, num_subcores=NS)
    gather_run = pl.kernel(
        _sc_gather_body,
        out_type=jax.ShapeDtypeStruct((D, N), jnp.float32),
        mesh=mesh,
        scratch_types=[
            pltpu.VMEM((N,), jnp.int32),
            pltpu.VMEM((N,), jnp.float32),
            pltpu.VMEM((N,), jnp.float32),
            pltpu.VMEM((N,), jnp.float32),
            pltpu.VMEM((N,), jnp.float32),
            pltpu.SemaphoreType.DMA,
            pltpu.SemaphoreType.DMA,
            pltpu.SemaphoreType.DMA,
            pltpu.SemaphoreType.DMA,
        ],
        compiler_params=pltpu.CompilerParams(
            use_tc_tiling_on_sc=True, needs_layout_passes=False),
    )
    xg_t = gather_run(xt, jnp.asarray(perm))

    out_t = pl.pallas_call(
        _tc_select_body,
        out_shape=jax.ShapeDtypeStruct((D, N), jnp.float32),
        grid=(N // BL,),
        in_specs=[
            pl.BlockSpec((D, BL), lambda i: (0, i)),
            pl.BlockSpec((BL,), lambda i: (i,)),
            pl.BlockSpec((D, BL), lambda i: (0, i)),
            pl.BlockSpec((D, BL), lambda i: (0, i)),
        ],
        out_specs=pl.BlockSpec((D, BL), lambda i: (0, i)),
        input_output_aliases={2: 0},
    )(jnp.asarray(ut), p, xg_t, xt)
    return out_t.T
